# Initial kernel scaffold; baseline (speedup 1.0000x reference)
import jax
import jax.numpy as jnp
from jax.experimental import pallas as pl


def kernel(x, edge_index, batch, W1, b1, W2, b2, Wfc, bfc):
    raise NotImplementedError("write your pallas kernel here")


# speedup vs baseline: 12.5738x; 12.5738x over previous
import functools

import jax
import jax.numpy as jnp
from jax import lax
from jax.experimental import pallas as pl
from jax.experimental.pallas import tpu as pltpu
from jax.experimental.pallas import tpu_sc as plsc

N = 10000
F = 128
G = 64
NPAD = 10240
NE = 320000
CHUNK = 128
NCORE = 2
NSUB = 16
NW = NCORE * NSUB
NCHUNK = 79
EPT = NCHUNK * CHUNK
NE_PAD = NW * EPT
ROWS_PER_TILE = NPAD // NSUB
ZB = 16

def _deg_body(dst_hbm, out_hbm, dstv, ones_v, zb, acc, sem):
    cid = lax.axis_index("c")
    sid = lax.axis_index("s")
    wid = sid * NCORE + cid

    def fill(i, _):
        ones_v[i, :] = jnp.ones((16,), jnp.float32)
        return 0

    lax.fori_loop(0, CHUNK, fill, 0)

    def zfill(i, _):
        zb[i, :] = jnp.zeros((16,), jnp.float32)
        return 0

    lax.fori_loop(0, ZB, zfill, 0)

    row0 = sid * ROWS_PER_TILE

    def zero_acc(i, _):
        pltpu.sync_copy(zb, acc.at[pl.ds(row0 + i * ZB, ZB)])
        return 0

    lax.fori_loop(0, ROWS_PER_TILE // ZB, zero_acc, 0)
    plsc.subcore_barrier()

    pltpu.sync_copy(dst_hbm.at[wid], dstv)

    def body(j, _):
        pltpu.sync_copy(ones_v, acc.at[dstv.at[j]], add=True)
        return 0

    lax.fori_loop(0, NCHUNK, body, 0)
    plsc.subcore_barrier()

    def wout(i, _):
        r = row0 + i * ZB
        pltpu.sync_copy(acc.at[pl.ds(r, ZB)], out_hbm.at[cid, pl.ds(r, ZB)])
        return 0

    lax.fori_loop(0, ROWS_PER_TILE // ZB, wout, 0)


def _scatter_body(h_hbm, src_hbm, dst_hbm, out_hbm, srcv, dstv, rows, zb, acc, sem):
    cid = lax.axis_index("c")
    sid = lax.axis_index("s")
    wid = sid * NCORE + cid

    def zfill(i, _):
        for j in range(F // 16):
            zb[i, pl.ds(j * 16, 16)] = jnp.zeros((16,), jnp.float32)
        return 0

    lax.fori_loop(0, ZB, zfill, 0)

    row0 = sid * ROWS_PER_TILE

    def zero_acc(i, _):
        pltpu.sync_copy(zb, acc.at[pl.ds(row0 + i * ZB, ZB)])
        return 0

    lax.fori_loop(0, ROWS_PER_TILE // ZB, zero_acc, 0)
    plsc.subcore_barrier()

    pltpu.sync_copy(src_hbm.at[wid], srcv)
    pltpu.sync_copy(dst_hbm.at[wid], dstv)

    def body(j, _):
        pltpu.async_copy(h_hbm.at[srcv.at[j]], rows, sem).wait()
        pltpu.sync_copy(rows, acc.at[dstv.at[j]], add=True)
        return 0

    lax.fori_loop(0, NCHUNK, body, 0)
    plsc.subcore_barrier()

    def wout(i, _):
        r = row0 + i * ZB
        pltpu.sync_copy(acc.at[pl.ds(r, ZB)], out_hbm.at[cid, pl.ds(r, ZB)])
        return 0

    lax.fori_loop(0, ROWS_PER_TILE // ZB, wout, 0)


@functools.cache
def _sc_kernels():
    mesh = plsc.VectorSubcoreMesh(
        core_axis_name="c", subcore_axis_name="s",
        num_cores=NCORE, num_subcores=NSUB,
    )
    deg = pl.kernel(
        _deg_body,
        out_type=jax.ShapeDtypeStruct((NCORE, NPAD, 16), jnp.float32),
        mesh=mesh,
        scratch_types=[
            pltpu.VMEM((NCHUNK, CHUNK), jnp.int32),
            pltpu.VMEM((CHUNK, 16), jnp.float32),
            pltpu.VMEM((ZB, 16), jnp.float32),
            pltpu.VMEM_SHARED((NPAD, 16), jnp.float32),
            pltpu.SemaphoreType.DMA,
        ],
    )
    scat = pl.kernel(
        _scatter_body,
        out_type=jax.ShapeDtypeStruct((NCORE, NPAD, F), jnp.float32),
        mesh=mesh,
        scratch_types=[
            pltpu.VMEM((NCHUNK, CHUNK), jnp.int32),
            pltpu.VMEM((NCHUNK, CHUNK), jnp.int32),
            pltpu.VMEM((CHUNK, F), jnp.float32),
            pltpu.VMEM((ZB, F), jnp.float32),
            pltpu.VMEM_SHARED((NPAD, F), jnp.float32),
            pltpu.SemaphoreType.DMA,
        ],
    )
    return deg, scat


_BLK = 1280


def _dinv_of(degp_ref):
    deg = degp_ref[0][:, 0:1] + degp_ref[1][:, 0:1] + 1.0
    return lax.rsqrt(deg)


def _tc1_body(x_ref, w_ref, degp_ref, o_ref):
    dinv = _dinv_of(degp_ref)
    o_ref[...] = (
        jnp.dot(x_ref[...], w_ref[...], preferred_element_type=jnp.float32) * dinv
    )


def _tc2_body(accp_ref, h_ref, degp_ref, w_ref, b_ref, o_ref):
    i = pl.program_id(0)
    dinv = _dinv_of(degp_ref)
    z = dinv * (accp_ref[0] + accp_ref[1] + h_ref[...]) + b_ref[...]
    z = jnp.maximum(z, 0.0)
    rows = i * _BLK + lax.broadcasted_iota(jnp.int32, (_BLK, 1), 0)
    z = jnp.where(rows < N, z, 0.0)
    o_ref[...] = (
        jnp.dot(z, w_ref[...], preferred_element_type=jnp.float32) * dinv
    )


def _tc3_body(accp_ref, h_ref, degp_ref, b_ref, batch_ref, wfc_ref, bfc_ref, o_ref):
    dinv = _dinv_of(degp_ref)
    z = dinv * (accp_ref[0] + accp_ref[1] + h_ref[...]) + b_ref[...]
    z = jnp.maximum(z, 0.0)
    oh = (batch_ref[...] == lax.broadcasted_iota(jnp.int32, (1, G), 1)).astype(
        jnp.float32
    )
    s = lax.dot_general(
        oh, z, (((0,), (0,)), ((), ())), preferred_element_type=jnp.float32
    )
    cnt = jnp.sum(oh, axis=0)[:, None]
    g = s / jnp.maximum(cnt, 1.0)
    o_ref[...] = jax.nn.sigmoid(
        jnp.dot(g, wfc_ref[...], preferred_element_type=jnp.float32) + bfc_ref[...]
    )


def _tc1(xpad, W1, degp):
    return pl.pallas_call(
        _tc1_body,
        grid=(NPAD // _BLK,),
        in_specs=[
            pl.BlockSpec((_BLK, F), lambda i: (i, 0)),
            pl.BlockSpec((F, F), lambda i: (0, 0)),
            pl.BlockSpec((NCORE, _BLK, 16), lambda i: (0, i, 0)),
        ],
        out_specs=pl.BlockSpec((_BLK, F), lambda i: (i, 0)),
        out_shape=jax.ShapeDtypeStruct((NPAD, F), jnp.float32),
    )(xpad, W1, degp)


def _tc2(accp, hpad, degp, W2, b1):
    return pl.pallas_call(
        _tc2_body,
        grid=(NPAD // _BLK,),
        in_specs=[
            pl.BlockSpec((NCORE, _BLK, F), lambda i: (0, i, 0)),
            pl.BlockSpec((_BLK, F), lambda i: (i, 0)),
            pl.BlockSpec((NCORE, _BLK, 16), lambda i: (0, i, 0)),
            pl.BlockSpec((F, F), lambda i: (0, 0)),
            pl.BlockSpec((1, F), lambda i: (0, 0)),
        ],
        out_specs=pl.BlockSpec((_BLK, F), lambda i: (i, 0)),
        out_shape=jax.ShapeDtypeStruct((NPAD, F), jnp.float32),
    )(accp, hpad, degp, W2, b1)


def _tc3(accp, hpad, degp, b2, batchp, Wfc, bfc):
    return pl.pallas_call(
        _tc3_body,
        grid=(1,),
        in_specs=[
            pl.BlockSpec((NCORE, NPAD, F), lambda i: (0, 0, 0)),
            pl.BlockSpec((NPAD, F), lambda i: (0, 0)),
            pl.BlockSpec((NCORE, NPAD, 16), lambda i: (0, 0, 0)),
            pl.BlockSpec((1, F), lambda i: (0, 0)),
            pl.BlockSpec((NPAD, 1), lambda i: (0, 0)),
            pl.BlockSpec((F, 16), lambda i: (0, 0)),
            pl.BlockSpec((1, 16), lambda i: (0, 0)),
        ],
        out_specs=pl.BlockSpec((G, 16), lambda i: (0, 0)),
        out_shape=jax.ShapeDtypeStruct((G, 16), jnp.float32),
    )(accp, hpad, degp, b2, batchp, Wfc, bfc)


def kernel(x, edge_index, batch, W1, b1, W2, b2, Wfc, bfc):
    src = edge_index[0].astype(jnp.int32)
    dst = edge_index[1].astype(jnp.int32)
    pad = jnp.full((NE_PAD - NE,), N, jnp.int32)
    src3 = jnp.concatenate([src, pad]).reshape(NW, NCHUNK, CHUNK)
    dst3 = jnp.concatenate([dst, pad]).reshape(NW, NCHUNK, CHUNK)

    xpad = jnp.pad(x, ((0, NPAD - N), (0, 0)))
    batchp = jnp.pad(
        batch.astype(jnp.int32), (0, NPAD - N), constant_values=G
    ).reshape(NPAD, 1)

    deg_kernel, scatter_kernel = _sc_kernels()
    degp = deg_kernel(dst3)
    h1 = _tc1(xpad, W1, degp)
    acc1 = scatter_kernel(h1, src3, dst3)
    h2 = _tc2(acc1, h1, degp, W2, b1.reshape(1, F))
    acc2 = scatter_kernel(h2, src3, dst3)
    return _tc3(acc2, h2, degp, b2.reshape(1, F), batchp, Wfc, bfc.reshape(1, 16))
